# baseline (speedup 1.0000x reference)
import functools

import jax
import jax.numpy as jnp
from jax import lax
from jax.experimental import pallas as pl
from jax.experimental.pallas import tpu as pltpu
from jax.experimental.pallas import tpu_sc as plsc

N = 10000
E = 320000
B = 256
D = 128
CODE = 256
FINAL = 128

NPAD = 10240
BR = 512
NB = NPAD // BR

NC = 2
NS = 16
CHUNK = 128
NCHUNK = 80
NPAIR = NCHUNK // 2
EPAD = NC * NS * NCHUNK * CHUNK
RPT = NPAD // NS
DEGW = 16

_HIGH = jax.lax.Precision.HIGHEST


def _mesh():
    return plsc.VectorSubcoreMesh(core_axis_name="c", subcore_axis_name="s")


@functools.cache
def _make_deg_kernel():
    return functools.partial(
        pl.kernel,
        mesh=_mesh(),
        out_type=jax.ShapeDtypeStruct((NC, NPAD, D), jnp.float32),
        scratch_types=[
            pltpu.VMEM((NCHUNK, CHUNK), jnp.int32),
            pltpu.VMEM((CHUNK, D), jnp.float32),
            pltpu.VMEM_SHARED((NPAD, D), jnp.float32),
            pltpu.SemaphoreType.DMA,
        ],
    )(_deg_body)


def _deg_body(dst_hbm, deg_out, dsts_v, ones_v, acc_s, sem):
    cid = lax.axis_index("c")
    sid = lax.axis_index("s")

    def zfill(i, _):
        for j in range(D // 16):
            ones_v[i, pl.ds(j * 16, 16)] = jnp.zeros((16,), jnp.float32)
        return 0

    lax.fori_loop(0, CHUNK, zfill, 0)
    pltpu.sync_copy(dst_hbm.at[cid, sid], dsts_v)
    for r in range(RPT // CHUNK):
        pltpu.sync_copy(ones_v, acc_s.at[pl.ds(sid * RPT + r * CHUNK, CHUNK)])

    def fill(i, _):
        for j in range(D // 16):
            ones_v[i, pl.ds(j * 16, 16)] = jnp.full((16,), 1.0, jnp.float32)
        return 0

    lax.fori_loop(0, CHUNK, fill, 0)
    plsc.subcore_barrier()

    GRP = 8

    def body(gi, _):
        for j in range(GRP):
            pltpu.async_copy(ones_v, acc_s.at[dsts_v.at[gi * GRP + j]],
                             sem, add=True)
        for j in range(GRP):
            pltpu.make_async_copy(ones_v, acc_s.at[dsts_v.at[gi * GRP + j]],
                                  sem).wait()
        return 0

    lax.fori_loop(0, NCHUNK // GRP, body, 0)
    plsc.subcore_barrier()
    pltpu.sync_copy(acc_s.at[pl.ds(sid * RPT, RPT)],
                    deg_out.at[cid, pl.ds(sid * RPT, RPT)])


@functools.cache
def _make_agg_kernel():
    return functools.partial(
        pl.kernel,
        mesh=_mesh(),
        out_type=jax.ShapeDtypeStruct((NC, NPAD, D), jnp.float32),
        scratch_types=[
            pltpu.VMEM((CHUNK,), jnp.int32),
            pltpu.VMEM((CHUNK,), jnp.int32),
            pltpu.VMEM((NCHUNK, CHUNK), jnp.int32),
            pltpu.VMEM((CHUNK, D), jnp.float32),
            pltpu.VMEM((CHUNK, D), jnp.float32),
            pltpu.VMEM_SHARED((NPAD, D), jnp.float32),
            pltpu.SemaphoreType.DMA,
            pltpu.SemaphoreType.DMA,
            pltpu.SemaphoreType.DMA,
            pltpu.SemaphoreType.DMA,
        ],
    )(_agg_body)


def _agg_body(g_hbm, src_hbm, dst_hbm, out_hbm, srcs0, srcs1, dsts_v,
              rows0, rows1, acc_s, sem0, sem1, isem0, isem1):
    cid = lax.axis_index("c")
    sid = lax.axis_index("s")

    def zfill(i, _):
        for j in range(D // 16):
            rows0[i, pl.ds(j * 16, 16)] = jnp.zeros((16,), jnp.float32)
        return 0

    lax.fori_loop(0, CHUNK, zfill, 0)
    pltpu.sync_copy(dst_hbm.at[cid, sid], dsts_v)
    for r in range(RPT // CHUNK):
        pltpu.sync_copy(rows0, acc_s.at[pl.ds(sid * RPT + r * CHUNK, CHUNK)])
    plsc.subcore_barrier()

    pltpu.sync_copy(src_hbm.at[cid, sid, 0], srcs0)
    pltpu.async_copy(g_hbm.at[srcs0], rows0, sem0)
    pltpu.async_copy(src_hbm.at[cid, sid, 1], srcs1, isem1)

    def body(p, _):
        a = 2 * p
        pltpu.make_async_copy(g_hbm.at[srcs0], rows0, sem0).wait()
        pltpu.make_async_copy(src_hbm.at[cid, sid, a + 1], srcs1, isem1).wait()
        pltpu.async_copy(g_hbm.at[srcs1], rows1, sem1)

        @pl.when(p < NPAIR - 1)
        def _pf0():
            pltpu.async_copy(src_hbm.at[cid, sid, a + 2], srcs0, isem0)

        pltpu.sync_copy(rows0, acc_s.at[dsts_v.at[a]], add=True)
        pltpu.make_async_copy(g_hbm.at[srcs1], rows1, sem1).wait()

        @pl.when(p < NPAIR - 1)
        def _next():
            pltpu.make_async_copy(src_hbm.at[cid, sid, a + 2], srcs0, isem0).wait()
            pltpu.async_copy(g_hbm.at[srcs0], rows0, sem0)
            pltpu.async_copy(src_hbm.at[cid, sid, a + 3], srcs1, isem1)

        pltpu.sync_copy(rows1, acc_s.at[dsts_v.at[a + 1]], add=True)
        return 0

    lax.fori_loop(0, NPAIR, body, 0)
    plsc.subcore_barrier()
    pltpu.sync_copy(acc_s.at[pl.ds(sid * RPT, RPT)],
                    out_hbm.at[cid, pl.ds(sid * RPT, RPT)])


def _tc1_body(x_ref, w_ref, deg_ref, g_ref, dinv_ref):
    d = deg_ref[0] + deg_ref[1]
    dtot = d[:, 0:1] + 1.0
    dinv = lax.rsqrt(jnp.maximum(dtot, 1.0))
    y = jnp.dot(x_ref[...], w_ref[...],
                preferred_element_type=jnp.float32, precision=_HIGH)
    g_ref[...] = y * dinv
    dinv_ref[...] = jnp.broadcast_to(dinv, (BR, DEGW))


def _tc1(x_pad, W1, degp):
    return pl.pallas_call(
        _tc1_body,
        grid=(NB,),
        in_specs=[
            pl.BlockSpec((BR, D), lambda k: (k, 0)),
            pl.BlockSpec((D, D), lambda k: (0, 0)),
            pl.BlockSpec((NC, BR, D), lambda k: (0, k, 0)),
        ],
        out_specs=[
            pl.BlockSpec((BR, D), lambda k: (k, 0)),
            pl.BlockSpec((BR, DEGW), lambda k: (k, 0)),
        ],
        out_shape=[
            jax.ShapeDtypeStruct((NPAD, D), jnp.float32),
            jax.ShapeDtypeStruct((NPAD, DEGW), jnp.float32),
        ],
    )(x_pad, W1, degp)


def _tc2_body(g_ref, p_ref, dinv_ref, b_ref, w_ref, o_ref):
    dinv = dinv_ref[:, 0:1]
    s = p_ref[0] + p_ref[1] + g_ref[...]
    h = jnp.maximum(s * dinv + b_ref[...], 0.0)
    y = jnp.dot(h, w_ref[...],
                preferred_element_type=jnp.float32, precision=_HIGH)
    o_ref[...] = y * dinv


def _tc2(g1, p1, dinv16, b1, W2):
    return pl.pallas_call(
        _tc2_body,
        grid=(NB,),
        in_specs=[
            pl.BlockSpec((BR, D), lambda k: (k, 0)),
            pl.BlockSpec((NC, BR, D), lambda k: (0, k, 0)),
            pl.BlockSpec((BR, DEGW), lambda k: (k, 0)),
            pl.BlockSpec((1, D), lambda k: (0, 0)),
            pl.BlockSpec((D, D), lambda k: (0, 0)),
        ],
        out_specs=pl.BlockSpec((BR, D), lambda k: (k, 0)),
        out_shape=jax.ShapeDtypeStruct((NPAD, D), jnp.float32),
    )(g1, p1, dinv16, b1, W2)


def _log_softmax(z):
    m = jnp.max(z, axis=-1, keepdims=True)
    zs = z - m
    return zs - jnp.log(jnp.sum(jnp.exp(zs), axis=-1, keepdims=True))


def _tc3_body(g_ref, p_ref, dinv_ref, b2_ref, batch_ref, cx_ref,
              wc_ref, bc_ref, wt_ref, bt_ref, wfc_ref, wft_ref, bf_ref,
              o1_ref, o2_ref, o3_ref, sums, counts):
    k = pl.program_id(0)

    @pl.when(k == 0)
    def _init():
        sums[...] = jnp.zeros_like(sums)
        counts[...] = jnp.zeros_like(counts)

    dinv = dinv_ref[:, 0:1]
    h2 = (p_ref[0] + p_ref[1] + g_ref[...]) * dinv + b2_ref[...]
    bvec = batch_ref[0]
    seg = lax.broadcasted_iota(jnp.int32, (B, BR), 0)
    onehot = (bvec == seg).astype(jnp.float32)
    sums[...] += jnp.dot(onehot, h2,
                         preferred_element_type=jnp.float32, precision=_HIGH)
    counts[...] += jnp.sum(onehot, axis=1, keepdims=True)

    @pl.when(k == NB - 1)
    def _final():
        cnt = jnp.maximum(counts[:, 0:1], 1.0)
        trans = sums[...] / cnt
        code = cx_ref[...]
        z1 = jnp.dot(code, wc_ref[...],
                     preferred_element_type=jnp.float32, precision=_HIGH) + bc_ref[...]
        o1_ref[...] = _log_softmax(z1)
        z2 = jnp.dot(trans, wt_ref[...],
                     preferred_element_type=jnp.float32, precision=_HIGH) + bt_ref[...]
        o2_ref[...] = _log_softmax(z2)
        z3 = (jnp.dot(code, wfc_ref[...],
                      preferred_element_type=jnp.float32, precision=_HIGH)
              + jnp.dot(trans, wft_ref[...],
                        preferred_element_type=jnp.float32, precision=_HIGH)
              + bf_ref[...])
        o3_ref[...] = _log_softmax(z3)


def _tc3(g2, p2, dinv16, b2, batch3, code_x, Wc, bc, Wt, bt, Wfc, Wft, bf):
    return pl.pallas_call(
        _tc3_body,
        grid=(NB,),
        in_specs=[
            pl.BlockSpec((BR, D), lambda k: (k, 0)),
            pl.BlockSpec((NC, BR, D), lambda k: (0, k, 0)),
            pl.BlockSpec((BR, DEGW), lambda k: (k, 0)),
            pl.BlockSpec((1, D), lambda k: (0, 0)),
            pl.BlockSpec((1, 1, BR), lambda k: (k, 0, 0)),
            pl.BlockSpec((B, CODE), lambda k: (0, 0)),
            pl.BlockSpec((CODE, FINAL), lambda k: (0, 0)),
            pl.BlockSpec((1, FINAL), lambda k: (0, 0)),
            pl.BlockSpec((D, FINAL), lambda k: (0, 0)),
            pl.BlockSpec((1, FINAL), lambda k: (0, 0)),
            pl.BlockSpec((CODE, FINAL), lambda k: (0, 0)),
            pl.BlockSpec((D, FINAL), lambda k: (0, 0)),
            pl.BlockSpec((1, FINAL), lambda k: (0, 0)),
        ],
        out_specs=[
            pl.BlockSpec((B, FINAL), lambda k: (0, 0)),
            pl.BlockSpec((B, FINAL), lambda k: (0, 0)),
            pl.BlockSpec((B, FINAL), lambda k: (0, 0)),
        ],
        out_shape=[
            jax.ShapeDtypeStruct((B, FINAL), jnp.float32),
            jax.ShapeDtypeStruct((B, FINAL), jnp.float32),
            jax.ShapeDtypeStruct((B, FINAL), jnp.float32),
        ],
        scratch_shapes=[
            pltpu.VMEM((B, FINAL), jnp.float32),
            pltpu.VMEM((B, FINAL), jnp.float32),
        ],
    )(g2, p2, dinv16, b2, batch3, code_x, Wc, bc, Wt, bt, Wfc, Wft, bf)


def kernel(x, edge_index, batch, code_x, W1, b1, W2, b2, Wc, bc, Wt, bt, Wf, bf):
    x_pad = jnp.pad(x, ((0, NPAD - N), (0, 0)))
    src = edge_index[0].astype(jnp.int32)
    dst = edge_index[1].astype(jnp.int32)
    fill = jnp.full((EPAD - E,), NPAD - 1, jnp.int32)
    src3 = jnp.concatenate([src, fill]).reshape(NC, NS, NCHUNK, CHUNK)[::-1]
    dst3 = jnp.concatenate([dst, fill]).reshape(NC, NS, NCHUNK, CHUNK)[::-1]
    batch3 = jnp.concatenate(
        [batch.astype(jnp.int32), jnp.full((NPAD - N,), 1 << 20, jnp.int32)]
    ).reshape(NB, 1, BR)

    b1r = b1.reshape(1, D)
    b2r = b2.reshape(1, D)
    bcr = bc.reshape(1, FINAL)
    btr = bt.reshape(1, FINAL)
    bfr = bf.reshape(1, FINAL)
    Wfc = Wf[:CODE]
    Wft = Wf[CODE:]

    deg_kernel = _make_deg_kernel()
    agg_kernel = _make_agg_kernel()
    degp = deg_kernel(dst3)
    g1, dinv16 = _tc1(x_pad, W1, degp)
    p1 = agg_kernel(g1, src3, dst3)
    g2 = _tc2(g1, p1, dinv16, b1r, W2)
    p2 = agg_kernel(g2, src3, dst3)
    code_prob, trans_prob, final_prob = _tc3(
        g2, p2, dinv16, b2r, batch3, code_x, Wc, bcr, Wt, btr, Wfc, Wft, bfr)
    return (code_prob, trans_prob, final_prob)


# speedup vs baseline: 3.0036x; 3.0036x over previous
import functools

import jax
import jax.numpy as jnp
from jax import lax
from jax.experimental import pallas as pl
from jax.experimental.pallas import tpu as pltpu
from jax.experimental.pallas import tpu_sc as plsc

N = 10000
E = 320000
B = 256
D = 128
CODE = 256
FINAL = 128

NPAD = 10240
BR = 512
NB = NPAD // BR

NC = 2
NS = 16
CHUNK = 128
NCHUNK = 80
NPAIR = NCHUNK // 2
EPAD = NC * NS * NCHUNK * CHUNK
RPT = NPAD // NS
DEGW = 16

_HIGH = jax.lax.Precision.HIGHEST


def _mesh():
    return plsc.VectorSubcoreMesh(core_axis_name="c", subcore_axis_name="s")


@functools.cache
def _make_deg_kernel():
    return functools.partial(
        pl.kernel,
        mesh=_mesh(),
        out_type=jax.ShapeDtypeStruct((NC, NPAD, D), jnp.float32),
        scratch_types=[
            pltpu.VMEM((NCHUNK, CHUNK), jnp.int32),
            pltpu.VMEM((CHUNK, D), jnp.float32),
            pltpu.VMEM_SHARED((NPAD, D), jnp.float32),
            pltpu.SemaphoreType.DMA,
        ],
    )(_deg_body)


def _deg_body(dst_hbm, deg_out, dsts_v, ones_v, acc_s, sem):
    cid = lax.axis_index("c")
    sid = lax.axis_index("s")

    def zfill(i, _):
        for j in range(D // 16):
            ones_v[i, pl.ds(j * 16, 16)] = jnp.zeros((16,), jnp.float32)
        return 0

    lax.fori_loop(0, CHUNK, zfill, 0)
    pltpu.sync_copy(dst_hbm.at[cid, sid], dsts_v)
    for r in range(RPT // CHUNK):
        pltpu.sync_copy(ones_v, acc_s.at[pl.ds(sid * RPT + r * CHUNK, CHUNK)])

    def fill(i, _):
        for j in range(D // 16):
            ones_v[i, pl.ds(j * 16, 16)] = jnp.full((16,), 1.0, jnp.float32)
        return 0

    lax.fori_loop(0, CHUNK, fill, 0)
    plsc.subcore_barrier()

    GRP = 8

    def body(gi, _):
        for j in range(GRP):
            pltpu.async_copy(ones_v, acc_s.at[dsts_v.at[gi * GRP + j]],
                             sem, add=True)
        for j in range(GRP):
            pltpu.make_async_copy(ones_v, acc_s.at[dsts_v.at[gi * GRP + j]],
                                  sem).wait()
        return 0

    lax.fori_loop(0, NCHUNK // GRP, body, 0)
    plsc.subcore_barrier()
    pltpu.sync_copy(acc_s.at[pl.ds(sid * RPT, RPT)],
                    deg_out.at[cid, pl.ds(sid * RPT, RPT)])


@functools.cache
def _make_agg_kernel():
    return functools.partial(
        pl.kernel,
        mesh=_mesh(),
        out_type=jax.ShapeDtypeStruct((NC, NPAD, D), jnp.float32),
        scratch_types=[
            pltpu.VMEM((CHUNK,), jnp.int32),
            pltpu.VMEM((CHUNK,), jnp.int32),
            pltpu.VMEM((NCHUNK, CHUNK), jnp.int32),
            pltpu.VMEM((CHUNK, D), jnp.float32),
            pltpu.VMEM((CHUNK, D), jnp.float32),
            pltpu.VMEM_SHARED((NPAD, D), jnp.float32),
            pltpu.SemaphoreType.DMA,
            pltpu.SemaphoreType.DMA,
            pltpu.SemaphoreType.DMA,
            pltpu.SemaphoreType.DMA,
        ],
    )(_agg_body)


def _agg_body(g_hbm, src_hbm, dst_hbm, out_hbm, srcs0, srcs1, dsts_v,
              rows0, rows1, acc_s, sem0, sem1, isem0, isem1):
    cid = lax.axis_index("c")
    sid = lax.axis_index("s")

    def zfill(i, _):
        for j in range(D // 16):
            rows0[i, pl.ds(j * 16, 16)] = jnp.zeros((16,), jnp.float32)
        return 0

    lax.fori_loop(0, CHUNK, zfill, 0)
    pltpu.sync_copy(dst_hbm.at[cid, sid], dsts_v)
    for r in range(RPT // CHUNK):
        pltpu.sync_copy(rows0, acc_s.at[pl.ds(sid * RPT + r * CHUNK, CHUNK)])
    plsc.subcore_barrier()

    pltpu.sync_copy(src_hbm.at[cid, sid, 0], srcs0)
    pltpu.async_copy(g_hbm.at[srcs0], rows0, sem0)
    pltpu.async_copy(src_hbm.at[cid, sid, 1], srcs1, isem1)

    def body(p, _):
        a = 2 * p
        pltpu.make_async_copy(g_hbm.at[srcs0], rows0, sem0).wait()
        pltpu.make_async_copy(src_hbm.at[cid, sid, a + 1], srcs1, isem1).wait()
        pltpu.async_copy(g_hbm.at[srcs1], rows1, sem1)

        @pl.when(p < NPAIR - 1)
        def _pf0():
            pltpu.async_copy(src_hbm.at[cid, sid, a + 2], srcs0, isem0)

        pltpu.sync_copy(rows0, acc_s.at[dsts_v.at[a]], add=True)
        pltpu.make_async_copy(g_hbm.at[srcs1], rows1, sem1).wait()

        @pl.when(p < NPAIR - 1)
        def _next():
            pltpu.make_async_copy(src_hbm.at[cid, sid, a + 2], srcs0, isem0).wait()
            pltpu.async_copy(g_hbm.at[srcs0], rows0, sem0)
            pltpu.async_copy(src_hbm.at[cid, sid, a + 3], srcs1, isem1)

        pltpu.sync_copy(rows1, acc_s.at[dsts_v.at[a + 1]], add=True)
        return 0

    lax.fori_loop(0, NPAIR, body, 0)
    plsc.subcore_barrier()
    pltpu.sync_copy(acc_s.at[pl.ds(sid * RPT, RPT)],
                    out_hbm.at[cid, pl.ds(sid * RPT, RPT)])


def _tc1_body(x_ref, w_ref, deg_ref, g_ref, dinv_ref):
    d = deg_ref[0] + deg_ref[1]
    dtot = d[:, 0:1] + 1.0
    dinv = lax.rsqrt(jnp.maximum(dtot, 1.0))
    y = jnp.dot(x_ref[...], w_ref[...],
                preferred_element_type=jnp.float32, precision=_HIGH)
    g_ref[...] = y * dinv
    dinv_ref[...] = jnp.broadcast_to(dinv, (BR, DEGW))


def _tc1(x_pad, W1, degp):
    return pl.pallas_call(
        _tc1_body,
        grid=(NB,),
        in_specs=[
            pl.BlockSpec((BR, D), lambda k: (k, 0)),
            pl.BlockSpec((D, D), lambda k: (0, 0)),
            pl.BlockSpec((NC, BR, D), lambda k: (0, k, 0)),
        ],
        out_specs=[
            pl.BlockSpec((BR, D), lambda k: (k, 0)),
            pl.BlockSpec((BR, DEGW), lambda k: (k, 0)),
        ],
        out_shape=[
            jax.ShapeDtypeStruct((NPAD, D), jnp.float32),
            jax.ShapeDtypeStruct((NPAD, DEGW), jnp.float32),
        ],
    )(x_pad, W1, degp)


def _tc2_body(g_ref, p_ref, dinv_ref, b_ref, w_ref, o_ref):
    dinv = dinv_ref[:, 0:1]
    s = p_ref[0] + p_ref[1] + g_ref[...]
    h = jnp.maximum(s * dinv + b_ref[...], 0.0)
    y = jnp.dot(h, w_ref[...],
                preferred_element_type=jnp.float32, precision=_HIGH)
    o_ref[...] = y * dinv


def _tc2(g1, p1, dinv16, b1, W2):
    return pl.pallas_call(
        _tc2_body,
        grid=(NB,),
        in_specs=[
            pl.BlockSpec((BR, D), lambda k: (k, 0)),
            pl.BlockSpec((NC, BR, D), lambda k: (0, k, 0)),
            pl.BlockSpec((BR, DEGW), lambda k: (k, 0)),
            pl.BlockSpec((1, D), lambda k: (0, 0)),
            pl.BlockSpec((D, D), lambda k: (0, 0)),
        ],
        out_specs=pl.BlockSpec((BR, D), lambda k: (k, 0)),
        out_shape=jax.ShapeDtypeStruct((NPAD, D), jnp.float32),
    )(g1, p1, dinv16, b1, W2)


def _log_softmax(z):
    m = jnp.max(z, axis=-1, keepdims=True)
    zs = z - m
    return zs - jnp.log(jnp.sum(jnp.exp(zs), axis=-1, keepdims=True))


def _tc3_body(g_ref, p_ref, dinv_ref, b2_ref, batch_ref, cx_ref,
              wc_ref, bc_ref, wt_ref, bt_ref, wfc_ref, wft_ref, bf_ref,
              o1_ref, o2_ref, o3_ref, sums, counts):
    k = pl.program_id(0)

    @pl.when(k == 0)
    def _init():
        sums[...] = jnp.zeros_like(sums)
        counts[...] = jnp.zeros_like(counts)

    dinv = dinv_ref[:, 0:1]
    h2 = (p_ref[0] + p_ref[1] + g_ref[...]) * dinv + b2_ref[...]
    bvec = batch_ref[0]
    seg = lax.broadcasted_iota(jnp.int32, (B, BR), 0)
    onehot = (bvec == seg).astype(jnp.float32)
    sums[...] += jnp.dot(onehot, h2,
                         preferred_element_type=jnp.float32, precision=_HIGH)
    counts[...] += jnp.sum(onehot, axis=1, keepdims=True)

    @pl.when(k == NB - 1)
    def _final():
        cnt = jnp.maximum(counts[:, 0:1], 1.0)
        trans = sums[...] / cnt
        code = cx_ref[...]
        z1 = jnp.dot(code, wc_ref[...],
                     preferred_element_type=jnp.float32, precision=_HIGH) + bc_ref[...]
        o1_ref[...] = _log_softmax(z1)
        z2 = jnp.dot(trans, wt_ref[...],
                     preferred_element_type=jnp.float32, precision=_HIGH) + bt_ref[...]
        o2_ref[...] = _log_softmax(z2)
        z3 = (jnp.dot(code, wfc_ref[...],
                      preferred_element_type=jnp.float32, precision=_HIGH)
              + jnp.dot(trans, wft_ref[...],
                        preferred_element_type=jnp.float32, precision=_HIGH)
              + bf_ref[...])
        o3_ref[...] = _log_softmax(z3)


def _tc3(g2, p2, dinv16, b2, batch3, code_x, Wc, bc, Wt, bt, Wfc, Wft, bf):
    return pl.pallas_call(
        _tc3_body,
        grid=(NB,),
        in_specs=[
            pl.BlockSpec((BR, D), lambda k: (k, 0)),
            pl.BlockSpec((NC, BR, D), lambda k: (0, k, 0)),
            pl.BlockSpec((BR, DEGW), lambda k: (k, 0)),
            pl.BlockSpec((1, D), lambda k: (0, 0)),
            pl.BlockSpec((1, 1, BR), lambda k: (k, 0, 0)),
            pl.BlockSpec((B, CODE), lambda k: (0, 0)),
            pl.BlockSpec((CODE, FINAL), lambda k: (0, 0)),
            pl.BlockSpec((1, FINAL), lambda k: (0, 0)),
            pl.BlockSpec((D, FINAL), lambda k: (0, 0)),
            pl.BlockSpec((1, FINAL), lambda k: (0, 0)),
            pl.BlockSpec((CODE, FINAL), lambda k: (0, 0)),
            pl.BlockSpec((D, FINAL), lambda k: (0, 0)),
            pl.BlockSpec((1, FINAL), lambda k: (0, 0)),
        ],
        out_specs=[
            pl.BlockSpec((B, FINAL), lambda k: (0, 0)),
            pl.BlockSpec((B, FINAL), lambda k: (0, 0)),
            pl.BlockSpec((B, FINAL), lambda k: (0, 0)),
        ],
        out_shape=[
            jax.ShapeDtypeStruct((B, FINAL), jnp.float32),
            jax.ShapeDtypeStruct((B, FINAL), jnp.float32),
            jax.ShapeDtypeStruct((B, FINAL), jnp.float32),
        ],
        scratch_shapes=[
            pltpu.VMEM((B, FINAL), jnp.float32),
            pltpu.VMEM((B, FINAL), jnp.float32),
        ],
    )(g2, p2, dinv16, b2, batch3, code_x, Wc, bc, Wt, bt, Wfc, Wft, bf)


def kernel(x, edge_index, batch, code_x, W1, b1, W2, b2, Wc, bc, Wt, bt, Wf, bf):
    x_pad = jnp.pad(x, ((0, NPAD - N), (0, 0)))
    src = edge_index[0].astype(jnp.int32)
    dst = edge_index[1].astype(jnp.int32)
    fill = N + (jnp.arange(EPAD - E, dtype=jnp.int32) % (NPAD - N))
    src3 = jnp.concatenate([src, fill]).reshape(NC, NS, NCHUNK, CHUNK)
    dst3 = jnp.concatenate([dst, fill]).reshape(NC, NS, NCHUNK, CHUNK)
    batch3 = jnp.concatenate(
        [batch.astype(jnp.int32), jnp.full((NPAD - N,), 1 << 20, jnp.int32)]
    ).reshape(NB, 1, BR)

    b1r = b1.reshape(1, D)
    b2r = b2.reshape(1, D)
    bcr = bc.reshape(1, FINAL)
    btr = bt.reshape(1, FINAL)
    bfr = bf.reshape(1, FINAL)
    Wfc = Wf[:CODE]
    Wft = Wf[CODE:]

    deg_kernel = _make_deg_kernel()
    agg_kernel = _make_agg_kernel()
    degp = deg_kernel(dst3)
    g1, dinv16 = _tc1(x_pad, W1, degp)
    p1 = agg_kernel(g1, src3, dst3)
    g2 = _tc2(g1, p1, dinv16, b1r, W2)
    p2 = agg_kernel(g2, src3, dst3)
    code_prob, trans_prob, final_prob = _tc3(
        g2, p2, dinv16, b2r, batch3, code_x, Wc, bcr, Wt, btr, Wfc, Wft, bfr)
    return (code_prob, trans_prob, final_prob)


# speedup vs baseline: 3.4655x; 1.1538x over previous
import functools

import jax
import jax.numpy as jnp
from jax import lax
from jax.experimental import pallas as pl
from jax.experimental.pallas import tpu as pltpu
from jax.experimental.pallas import tpu_sc as plsc

N = 10000
E = 320000
B = 256
D = 128
CODE = 256
FINAL = 128

NPAD = 10240
BR = 512
NB = NPAD // BR

NC = 2
NS = 16
CHUNK = 128
NCHUNK = 80
CH = 64
NCH = 160
NRING = 4
NOUT = NCH // NRING
EPAD = NC * NS * NCHUNK * CHUNK
RPT = NPAD // NS
DEGW = 16

_HIGH = jax.lax.Precision.HIGHEST


def _mesh():
    return plsc.VectorSubcoreMesh(core_axis_name="c", subcore_axis_name="s")


@functools.cache
def _make_deg_kernel():
    return functools.partial(
        pl.kernel,
        mesh=_mesh(),
        out_type=jax.ShapeDtypeStruct((NC, NPAD, D), jnp.float32),
        scratch_types=[
            pltpu.VMEM((NCHUNK, CHUNK), jnp.int32),
            pltpu.VMEM((CHUNK, D), jnp.float32),
            pltpu.VMEM_SHARED((NPAD, D), jnp.float32),
            pltpu.SemaphoreType.DMA,
        ],
    )(_deg_body)


def _deg_body(dst_hbm, deg_out, dsts_v, ones_v, acc_s, sem):
    cid = lax.axis_index("c")
    sid = lax.axis_index("s")

    def zfill(i, _):
        for j in range(D // 16):
            ones_v[i, pl.ds(j * 16, 16)] = jnp.zeros((16,), jnp.float32)
        return 0

    lax.fori_loop(0, CHUNK, zfill, 0)
    pltpu.sync_copy(dst_hbm.at[cid, sid], dsts_v)
    for r in range(RPT // CHUNK):
        pltpu.sync_copy(ones_v, acc_s.at[pl.ds(sid * RPT + r * CHUNK, CHUNK)])

    def fill(i, _):
        for j in range(D // 16):
            ones_v[i, pl.ds(j * 16, 16)] = jnp.full((16,), 1.0, jnp.float32)
        return 0

    lax.fori_loop(0, CHUNK, fill, 0)
    plsc.subcore_barrier()

    GRP = 8

    def body(gi, _):
        for j in range(GRP):
            pltpu.async_copy(ones_v, acc_s.at[dsts_v.at[gi * GRP + j]],
                             sem, add=True)
        for j in range(GRP):
            pltpu.make_async_copy(ones_v, acc_s.at[dsts_v.at[gi * GRP + j]],
                                  sem).wait()
        return 0

    lax.fori_loop(0, NCHUNK // GRP, body, 0)
    plsc.subcore_barrier()
    pltpu.sync_copy(acc_s.at[pl.ds(sid * RPT, RPT)],
                    deg_out.at[cid, pl.ds(sid * RPT, RPT)])


@functools.cache
def _make_agg_kernel():
    return functools.partial(
        pl.kernel,
        mesh=_mesh(),
        out_type=jax.ShapeDtypeStruct((NC, NPAD, D), jnp.float32),
        scratch_types=[
            pltpu.VMEM((NRING, CH), jnp.int32),
            pltpu.VMEM((NRING, CH), jnp.int32),
            pltpu.VMEM((NRING, CH, D), jnp.float32),
            pltpu.VMEM_SHARED((NPAD, D), jnp.float32),
        ]
        + [pltpu.SemaphoreType.DMA] * (4 * NRING),
    )(_agg_body)


def _agg_body(g_hbm, src_hbm, dst_hbm, out_hbm, isrc, idst, rows, acc_s,
              *sems):
    gsem = sems[0:NRING]
    ssem = sems[NRING:2 * NRING]
    isem = sems[2 * NRING:3 * NRING]
    dsem = sems[3 * NRING:4 * NRING]
    cid = lax.axis_index("c")
    sid = lax.axis_index("s")

    def zfill(i, _):
        for j in range(D // 16):
            rows[0, i, pl.ds(j * 16, 16)] = jnp.zeros((16,), jnp.float32)
        return 0

    lax.fori_loop(0, CH, zfill, 0)
    for r in range(RPT // CH):
        pltpu.sync_copy(rows.at[0], acc_s.at[pl.ds(sid * RPT + r * CH, CH)])
    plsc.subcore_barrier()

    for j in range(NRING):
        pltpu.async_copy(src_hbm.at[cid, sid, j], isrc.at[j], isem[j])
    for j in range(NRING - 1):
        pltpu.async_copy(dst_hbm.at[cid, sid, j], idst.at[j], dsem[j])
        pltpu.make_async_copy(src_hbm.at[cid, sid, j], isrc.at[j],
                              isem[j]).wait()
        pltpu.async_copy(g_hbm.at[isrc.at[j]], rows.at[j], gsem[j])

    def body(o, _):
        for j in range(NRING):
            c = NRING * o + j
            b3 = (j + 3) % NRING
            pltpu.make_async_copy(g_hbm.at[isrc.at[j]], rows.at[j],
                                  gsem[j]).wait()
            pltpu.make_async_copy(dst_hbm.at[cid, sid, c], idst.at[j],
                                  dsem[j]).wait()
            pltpu.async_copy(rows.at[j], acc_s.at[idst.at[j]], ssem[j],
                             add=True)

            @pl.when(o < NOUT - 1)
            def _pf():
                pltpu.async_copy(src_hbm.at[cid, sid, c + NRING],
                                 isrc.at[j], isem[j])

            def _recycle():
                pltpu.make_async_copy(rows.at[b3], acc_s.at[idst.at[b3]],
                                      ssem[b3]).wait()
                pltpu.make_async_copy(src_hbm.at[cid, sid, c], isrc.at[b3],
                                      isem[b3]).wait()
                pltpu.async_copy(g_hbm.at[isrc.at[b3]], rows.at[b3],
                                 gsem[b3])
                pltpu.async_copy(dst_hbm.at[cid, sid, c + 3], idst.at[b3],
                                 dsem[b3])

            if j == 0:
                @pl.when(o > 0)
                def _w0():
                    pltpu.make_async_copy(rows.at[b3], acc_s.at[idst.at[b3]],
                                          ssem[b3]).wait()

                pltpu.make_async_copy(src_hbm.at[cid, sid, c], isrc.at[b3],
                                      isem[b3]).wait()
                pltpu.async_copy(g_hbm.at[isrc.at[b3]], rows.at[b3],
                                 gsem[b3])
                pltpu.async_copy(dst_hbm.at[cid, sid, c + 3], idst.at[b3],
                                 dsem[b3])
            else:
                @pl.when(o < NOUT - 1)
                def _wj():
                    _recycle()

        return 0

    lax.fori_loop(0, NOUT, body, 0)
    for j in range(NRING):
        pltpu.make_async_copy(rows.at[j], acc_s.at[idst.at[j]],
                              ssem[j]).wait()
    plsc.subcore_barrier()
    pltpu.sync_copy(acc_s.at[pl.ds(sid * RPT, RPT)],
                    out_hbm.at[cid, pl.ds(sid * RPT, RPT)])


def _tc1_body(x_ref, w_ref, deg_ref, g_ref, dinv_ref):
    d = deg_ref[0] + deg_ref[1]
    dtot = d[:, 0:1] + 1.0
    dinv = lax.rsqrt(jnp.maximum(dtot, 1.0))
    y = jnp.dot(x_ref[...], w_ref[...],
                preferred_element_type=jnp.float32, precision=_HIGH)
    g_ref[...] = y * dinv
    dinv_ref[...] = jnp.broadcast_to(dinv, (BR, DEGW))


def _tc1(x_pad, W1, degp):
    return pl.pallas_call(
        _tc1_body,
        grid=(NB,),
        in_specs=[
            pl.BlockSpec((BR, D), lambda k: (k, 0)),
            pl.BlockSpec((D, D), lambda k: (0, 0)),
            pl.BlockSpec((NC, BR, D), lambda k: (0, k, 0)),
        ],
        out_specs=[
            pl.BlockSpec((BR, D), lambda k: (k, 0)),
            pl.BlockSpec((BR, DEGW), lambda k: (k, 0)),
        ],
        out_shape=[
            jax.ShapeDtypeStruct((NPAD, D), jnp.float32),
            jax.ShapeDtypeStruct((NPAD, DEGW), jnp.float32),
        ],
    )(x_pad, W1, degp)


def _tc2_body(g_ref, p_ref, dinv_ref, b_ref, w_ref, o_ref):
    dinv = dinv_ref[:, 0:1]
    s = p_ref[0] + p_ref[1] + g_ref[...]
    h = jnp.maximum(s * dinv + b_ref[...], 0.0)
    y = jnp.dot(h, w_ref[...],
                preferred_element_type=jnp.float32, precision=_HIGH)
    o_ref[...] = y * dinv


def _tc2(g1, p1, dinv16, b1, W2):
    return pl.pallas_call(
        _tc2_body,
        grid=(NB,),
        in_specs=[
            pl.BlockSpec((BR, D), lambda k: (k, 0)),
            pl.BlockSpec((NC, BR, D), lambda k: (0, k, 0)),
            pl.BlockSpec((BR, DEGW), lambda k: (k, 0)),
            pl.BlockSpec((1, D), lambda k: (0, 0)),
            pl.BlockSpec((D, D), lambda k: (0, 0)),
        ],
        out_specs=pl.BlockSpec((BR, D), lambda k: (k, 0)),
        out_shape=jax.ShapeDtypeStruct((NPAD, D), jnp.float32),
    )(g1, p1, dinv16, b1, W2)


def _log_softmax(z):
    m = jnp.max(z, axis=-1, keepdims=True)
    zs = z - m
    return zs - jnp.log(jnp.sum(jnp.exp(zs), axis=-1, keepdims=True))


def _tc3_body(g_ref, p_ref, dinv_ref, b2_ref, batch_ref, cx_ref,
              wc_ref, bc_ref, wt_ref, bt_ref, wfc_ref, wft_ref, bf_ref,
              o1_ref, o2_ref, o3_ref, sums, counts):
    k = pl.program_id(0)

    @pl.when(k == 0)
    def _init():
        sums[...] = jnp.zeros_like(sums)
        counts[...] = jnp.zeros_like(counts)

    dinv = dinv_ref[:, 0:1]
    h2 = (p_ref[0] + p_ref[1] + g_ref[...]) * dinv + b2_ref[...]
    bvec = batch_ref[0]
    seg = lax.broadcasted_iota(jnp.int32, (B, BR), 0)
    onehot = (bvec == seg).astype(jnp.float32)
    sums[...] += jnp.dot(onehot, h2,
                         preferred_element_type=jnp.float32, precision=_HIGH)
    counts[...] += jnp.sum(onehot, axis=1, keepdims=True)

    @pl.when(k == NB - 1)
    def _final():
        cnt = jnp.maximum(counts[:, 0:1], 1.0)
        trans = sums[...] / cnt
        code = cx_ref[...]
        z1 = jnp.dot(code, wc_ref[...],
                     preferred_element_type=jnp.float32, precision=_HIGH) + bc_ref[...]
        o1_ref[...] = _log_softmax(z1)
        z2 = jnp.dot(trans, wt_ref[...],
                     preferred_element_type=jnp.float32, precision=_HIGH) + bt_ref[...]
        o2_ref[...] = _log_softmax(z2)
        z3 = (jnp.dot(code, wfc_ref[...],
                      preferred_element_type=jnp.float32, precision=_HIGH)
              + jnp.dot(trans, wft_ref[...],
                        preferred_element_type=jnp.float32, precision=_HIGH)
              + bf_ref[...])
        o3_ref[...] = _log_softmax(z3)


def _tc3(g2, p2, dinv16, b2, batch3, code_x, Wc, bc, Wt, bt, Wfc, Wft, bf):
    return pl.pallas_call(
        _tc3_body,
        grid=(NB,),
        in_specs=[
            pl.BlockSpec((BR, D), lambda k: (k, 0)),
            pl.BlockSpec((NC, BR, D), lambda k: (0, k, 0)),
            pl.BlockSpec((BR, DEGW), lambda k: (k, 0)),
            pl.BlockSpec((1, D), lambda k: (0, 0)),
            pl.BlockSpec((1, 1, BR), lambda k: (k, 0, 0)),
            pl.BlockSpec((B, CODE), lambda k: (0, 0)),
            pl.BlockSpec((CODE, FINAL), lambda k: (0, 0)),
            pl.BlockSpec((1, FINAL), lambda k: (0, 0)),
            pl.BlockSpec((D, FINAL), lambda k: (0, 0)),
            pl.BlockSpec((1, FINAL), lambda k: (0, 0)),
            pl.BlockSpec((CODE, FINAL), lambda k: (0, 0)),
            pl.BlockSpec((D, FINAL), lambda k: (0, 0)),
            pl.BlockSpec((1, FINAL), lambda k: (0, 0)),
        ],
        out_specs=[
            pl.BlockSpec((B, FINAL), lambda k: (0, 0)),
            pl.BlockSpec((B, FINAL), lambda k: (0, 0)),
            pl.BlockSpec((B, FINAL), lambda k: (0, 0)),
        ],
        out_shape=[
            jax.ShapeDtypeStruct((B, FINAL), jnp.float32),
            jax.ShapeDtypeStruct((B, FINAL), jnp.float32),
            jax.ShapeDtypeStruct((B, FINAL), jnp.float32),
        ],
        scratch_shapes=[
            pltpu.VMEM((B, FINAL), jnp.float32),
            pltpu.VMEM((B, FINAL), jnp.float32),
        ],
    )(g2, p2, dinv16, b2, batch3, code_x, Wc, bc, Wt, bt, Wfc, Wft, bf)


def kernel(x, edge_index, batch, code_x, W1, b1, W2, b2, Wc, bc, Wt, bt, Wf, bf):
    x_pad = jnp.pad(x, ((0, NPAD - N), (0, 0)))
    src = edge_index[0].astype(jnp.int32)
    dst = edge_index[1].astype(jnp.int32)
    fill = N + (jnp.arange(EPAD - E, dtype=jnp.int32) % (NPAD - N))
    srcp = jnp.concatenate([src, fill])
    dstp = jnp.concatenate([dst, fill])
    src4 = srcp.reshape(NC, NS, NCH, CH)
    dst4 = dstp.reshape(NC, NS, NCH, CH)
    dst3 = dstp.reshape(NC, NS, NCHUNK, CHUNK)
    batch3 = jnp.concatenate(
        [batch.astype(jnp.int32), jnp.full((NPAD - N,), 1 << 20, jnp.int32)]
    ).reshape(NB, 1, BR)

    b1r = b1.reshape(1, D)
    b2r = b2.reshape(1, D)
    bcr = bc.reshape(1, FINAL)
    btr = bt.reshape(1, FINAL)
    bfr = bf.reshape(1, FINAL)
    Wfc = Wf[:CODE]
    Wft = Wf[CODE:]

    deg_kernel = _make_deg_kernel()
    agg_kernel = _make_agg_kernel()
    degp = deg_kernel(dst3)
    g1, dinv16 = _tc1(x_pad, W1, degp)
    p1 = agg_kernel(g1, src4, dst4)
    g2 = _tc2(g1, p1, dinv16, b1r, W2)
    p2 = agg_kernel(g2, src4, dst4)
    code_prob, trans_prob, final_prob = _tc3(
        g2, p2, dinv16, b2r, batch3, code_x, Wc, bcr, Wt, btr, Wfc, Wft, bfr)
    return (code_prob, trans_prob, final_prob)


# speedup vs baseline: 4.0043x; 1.1555x over previous
import functools

import jax
import jax.numpy as jnp
from jax import lax
from jax.experimental import pallas as pl
from jax.experimental.pallas import tpu as pltpu
from jax.experimental.pallas import tpu_sc as plsc

N = 10000
E = 320000
B = 256
D = 128
CODE = 256
FINAL = 128

NPAD = 10240
BR = 512
NB = NPAD // BR

NC = 2
NS = 16
CHUNK = 128
NCHUNK = 80
CH = 64
NCH = 160
NRING = 4
NOUT = NCH // NRING
EPAD = NC * NS * NCHUNK * CHUNK
RPT = NPAD // NS
DEGW = 16

_HIGH = jax.lax.Precision.HIGHEST


def _mesh():
    return plsc.VectorSubcoreMesh(core_axis_name="c", subcore_axis_name="s")


EPT = EPAD // (NC * NS)


@functools.cache
def _make_deg_kernel():
    return functools.partial(
        pl.kernel,
        mesh=_mesh(),
        out_type=jax.ShapeDtypeStruct((NC, NPAD), jnp.float32),
        compiler_params=pltpu.CompilerParams(needs_layout_passes=False),
        scratch_types=[
            pltpu.VMEM((EPT,), jnp.int32),
            pltpu.VMEM((NPAD,), jnp.float32),
            pltpu.VMEM((NS, RPT), jnp.float32),
            pltpu.VMEM_SHARED((NS, NPAD), jnp.float32),
            pltpu.SemaphoreType.DMA,
        ],
    )(_deg_body)


def _deg_body(dst_hbm, deg_out, idx_v, hist_v, tmp_v, hists_sh, sem):
    cid = lax.axis_index("c")
    sid = lax.axis_index("s")

    def zf(i, _):
        hist_v[pl.ds(i * 16, 16)] = jnp.zeros((16,), jnp.float32)
        return 0

    lax.fori_loop(0, NPAD // 16, zf, 0)
    pltpu.sync_copy(dst_hbm.at[cid, sid], idx_v)
    ones = jnp.full((16,), 1.0, jnp.float32)

    def body(k, _):
        for u in range(4):
            idx16 = idx_v[pl.ds((4 * k + u) * 16, 16)]
            plsc.addupdate_scatter(hist_v, [idx16], ones)
        return 0

    lax.fori_loop(0, EPT // 64, body, 0)
    pltpu.sync_copy(hist_v, hists_sh.at[sid])
    plsc.subcore_barrier()
    for t in range(NS):
        pltpu.async_copy(hists_sh.at[t, pl.ds(sid * RPT, RPT)],
                         tmp_v.at[t], sem)
    for t in range(NS):
        pltpu.make_async_copy(hists_sh.at[t, pl.ds(sid * RPT, RPT)],
                              tmp_v.at[t], sem).wait()

    def red(j, _):
        s = tmp_v[0, pl.ds(j * 16, 16)]
        for t in range(1, NS):
            s = s + tmp_v[t, pl.ds(j * 16, 16)]
        hist_v[pl.ds(j * 16, 16)] = s
        return 0

    lax.fori_loop(0, RPT // 16, red, 0)
    pltpu.sync_copy(hist_v.at[pl.ds(0, RPT)],
                    deg_out.at[cid, pl.ds(sid * RPT, RPT)])


@functools.cache
def _make_agg_kernel():
    return functools.partial(
        pl.kernel,
        mesh=_mesh(),
        out_type=jax.ShapeDtypeStruct((NC, NPAD, D), jnp.float32),
        scratch_types=[
            pltpu.VMEM((NRING, CH), jnp.int32),
            pltpu.VMEM((NRING, CH), jnp.int32),
            pltpu.VMEM((NRING, CH, D), jnp.float32),
            pltpu.VMEM_SHARED((NPAD, D), jnp.float32),
        ]
        + [pltpu.SemaphoreType.DMA] * (4 * NRING),
    )(_agg_body)


def _agg_body(g_hbm, src_hbm, dst_hbm, out_hbm, isrc, idst, rows, acc_s,
              *sems):
    gsem = sems[0:NRING]
    ssem = sems[NRING:2 * NRING]
    isem = sems[2 * NRING:3 * NRING]
    dsem = sems[3 * NRING:4 * NRING]
    cid = lax.axis_index("c")
    sid = lax.axis_index("s")

    def zfill(i, _):
        for j in range(D // 16):
            rows[0, i, pl.ds(j * 16, 16)] = jnp.zeros((16,), jnp.float32)
        return 0

    lax.fori_loop(0, CH, zfill, 0)
    for r in range(RPT // CH):
        pltpu.sync_copy(rows.at[0], acc_s.at[pl.ds(sid * RPT + r * CH, CH)])
    plsc.subcore_barrier()

    for j in range(NRING):
        pltpu.async_copy(src_hbm.at[cid, sid, j], isrc.at[j], isem[j])
    for j in range(NRING - 1):
        pltpu.async_copy(dst_hbm.at[cid, sid, j], idst.at[j], dsem[j])
        pltpu.make_async_copy(src_hbm.at[cid, sid, j], isrc.at[j],
                              isem[j]).wait()
        pltpu.async_copy(g_hbm.at[isrc.at[j]], rows.at[j], gsem[j])

    def body(o, _):
        for j in range(NRING):
            c = NRING * o + j
            b3 = (j + 3) % NRING
            pltpu.make_async_copy(g_hbm.at[isrc.at[j]], rows.at[j],
                                  gsem[j]).wait()
            pltpu.make_async_copy(dst_hbm.at[cid, sid, c], idst.at[j],
                                  dsem[j]).wait()
            pltpu.async_copy(rows.at[j], acc_s.at[idst.at[j]], ssem[j],
                             add=True)

            @pl.when(o < NOUT - 1)
            def _pf():
                pltpu.async_copy(src_hbm.at[cid, sid, c + NRING],
                                 isrc.at[j], isem[j])

            def _recycle():
                pltpu.make_async_copy(rows.at[b3], acc_s.at[idst.at[b3]],
                                      ssem[b3]).wait()
                pltpu.make_async_copy(src_hbm.at[cid, sid, c], isrc.at[b3],
                                      isem[b3]).wait()
                pltpu.async_copy(g_hbm.at[isrc.at[b3]], rows.at[b3],
                                 gsem[b3])
                pltpu.async_copy(dst_hbm.at[cid, sid, c + 3], idst.at[b3],
                                 dsem[b3])

            if j == 0:
                @pl.when(o > 0)
                def _w0():
                    pltpu.make_async_copy(rows.at[b3], acc_s.at[idst.at[b3]],
                                          ssem[b3]).wait()

                pltpu.make_async_copy(src_hbm.at[cid, sid, c], isrc.at[b3],
                                      isem[b3]).wait()
                pltpu.async_copy(g_hbm.at[isrc.at[b3]], rows.at[b3],
                                 gsem[b3])
                pltpu.async_copy(dst_hbm.at[cid, sid, c + 3], idst.at[b3],
                                 dsem[b3])
            else:
                @pl.when(o < NOUT - 1)
                def _wj():
                    _recycle()

        return 0

    lax.fori_loop(0, NOUT, body, 0)
    for j in range(NRING):
        pltpu.make_async_copy(rows.at[j], acc_s.at[idst.at[j]],
                              ssem[j]).wait()
    plsc.subcore_barrier()
    pltpu.sync_copy(acc_s.at[pl.ds(sid * RPT, RPT)],
                    out_hbm.at[cid, pl.ds(sid * RPT, RPT)])


def _tc1_body(x_ref, w_ref, deg_ref, g_ref, dinv_ref):
    dtot = deg_ref[0] + deg_ref[1] + 1.0
    dinv = lax.rsqrt(jnp.maximum(dtot, 1.0))
    y = jnp.dot(x_ref[...], w_ref[...],
                preferred_element_type=jnp.float32, precision=_HIGH)
    g_ref[...] = y * dinv
    dinv_ref[...] = dinv


def _tc1(x_pad, W1, degp):
    return pl.pallas_call(
        _tc1_body,
        grid=(NB,),
        in_specs=[
            pl.BlockSpec((BR, D), lambda k: (k, 0)),
            pl.BlockSpec((D, D), lambda k: (0, 0)),
            pl.BlockSpec((NC, BR, 1), lambda k: (0, k, 0)),
        ],
        out_specs=[
            pl.BlockSpec((BR, D), lambda k: (k, 0)),
            pl.BlockSpec((BR, 1), lambda k: (k, 0)),
        ],
        out_shape=[
            jax.ShapeDtypeStruct((NPAD, D), jnp.float32),
            jax.ShapeDtypeStruct((NPAD, 1), jnp.float32),
        ],
    )(x_pad, W1, degp)


def _tc2_body(g_ref, p_ref, dinv_ref, b_ref, w_ref, o_ref):
    dinv = dinv_ref[...]
    s = p_ref[0] + p_ref[1] + g_ref[...]
    h = jnp.maximum(s * dinv + b_ref[...], 0.0)
    y = jnp.dot(h, w_ref[...],
                preferred_element_type=jnp.float32, precision=_HIGH)
    o_ref[...] = y * dinv


def _tc2(g1, p1, dinv16, b1, W2):
    return pl.pallas_call(
        _tc2_body,
        grid=(NB,),
        in_specs=[
            pl.BlockSpec((BR, D), lambda k: (k, 0)),
            pl.BlockSpec((NC, BR, D), lambda k: (0, k, 0)),
            pl.BlockSpec((BR, 1), lambda k: (k, 0)),
            pl.BlockSpec((1, D), lambda k: (0, 0)),
            pl.BlockSpec((D, D), lambda k: (0, 0)),
        ],
        out_specs=pl.BlockSpec((BR, D), lambda k: (k, 0)),
        out_shape=jax.ShapeDtypeStruct((NPAD, D), jnp.float32),
    )(g1, p1, dinv16, b1, W2)


def _log_softmax(z):
    m = jnp.max(z, axis=-1, keepdims=True)
    zs = z - m
    return zs - jnp.log(jnp.sum(jnp.exp(zs), axis=-1, keepdims=True))


def _tc3_body(g_ref, p_ref, dinv_ref, b2_ref, batch_ref, cx_ref,
              wc_ref, bc_ref, wt_ref, bt_ref, wfc_ref, wft_ref, bf_ref,
              o1_ref, o2_ref, o3_ref, sums, counts):
    k = pl.program_id(0)

    @pl.when(k == 0)
    def _init():
        sums[...] = jnp.zeros_like(sums)
        counts[...] = jnp.zeros_like(counts)

    dinv = dinv_ref[...]
    h2 = (p_ref[0] + p_ref[1] + g_ref[...]) * dinv + b2_ref[...]
    bvec = batch_ref[0]
    seg = lax.broadcasted_iota(jnp.int32, (B, BR), 0)
    onehot = (bvec == seg).astype(jnp.float32)
    sums[...] += jnp.dot(onehot, h2,
                         preferred_element_type=jnp.float32, precision=_HIGH)
    counts[...] += jnp.sum(onehot, axis=1, keepdims=True)

    @pl.when(k == NB - 1)
    def _final():
        cnt = jnp.maximum(counts[:, 0:1], 1.0)
        trans = sums[...] / cnt
        code = cx_ref[...]
        z1 = jnp.dot(code, wc_ref[...],
                     preferred_element_type=jnp.float32, precision=_HIGH) + bc_ref[...]
        o1_ref[...] = _log_softmax(z1)
        z2 = jnp.dot(trans, wt_ref[...],
                     preferred_element_type=jnp.float32, precision=_HIGH) + bt_ref[...]
        o2_ref[...] = _log_softmax(z2)
        z3 = (jnp.dot(code, wfc_ref[...],
                      preferred_element_type=jnp.float32, precision=_HIGH)
              + jnp.dot(trans, wft_ref[...],
                        preferred_element_type=jnp.float32, precision=_HIGH)
              + bf_ref[...])
        o3_ref[...] = _log_softmax(z3)


def _tc3(g2, p2, dinv16, b2, batch3, code_x, Wc, bc, Wt, bt, Wfc, Wft, bf):
    return pl.pallas_call(
        _tc3_body,
        grid=(NB,),
        in_specs=[
            pl.BlockSpec((BR, D), lambda k: (k, 0)),
            pl.BlockSpec((NC, BR, D), lambda k: (0, k, 0)),
            pl.BlockSpec((BR, 1), lambda k: (k, 0)),
            pl.BlockSpec((1, D), lambda k: (0, 0)),
            pl.BlockSpec((1, 1, BR), lambda k: (k, 0, 0)),
            pl.BlockSpec((B, CODE), lambda k: (0, 0)),
            pl.BlockSpec((CODE, FINAL), lambda k: (0, 0)),
            pl.BlockSpec((1, FINAL), lambda k: (0, 0)),
            pl.BlockSpec((D, FINAL), lambda k: (0, 0)),
            pl.BlockSpec((1, FINAL), lambda k: (0, 0)),
            pl.BlockSpec((CODE, FINAL), lambda k: (0, 0)),
            pl.BlockSpec((D, FINAL), lambda k: (0, 0)),
            pl.BlockSpec((1, FINAL), lambda k: (0, 0)),
        ],
        out_specs=[
            pl.BlockSpec((B, FINAL), lambda k: (0, 0)),
            pl.BlockSpec((B, FINAL), lambda k: (0, 0)),
            pl.BlockSpec((B, FINAL), lambda k: (0, 0)),
        ],
        out_shape=[
            jax.ShapeDtypeStruct((B, FINAL), jnp.float32),
            jax.ShapeDtypeStruct((B, FINAL), jnp.float32),
            jax.ShapeDtypeStruct((B, FINAL), jnp.float32),
        ],
        scratch_shapes=[
            pltpu.VMEM((B, FINAL), jnp.float32),
            pltpu.VMEM((B, FINAL), jnp.float32),
        ],
    )(g2, p2, dinv16, b2, batch3, code_x, Wc, bc, Wt, bt, Wfc, Wft, bf)


def kernel(x, edge_index, batch, code_x, W1, b1, W2, b2, Wc, bc, Wt, bt, Wf, bf):
    x_pad = jnp.pad(x, ((0, NPAD - N), (0, 0)))
    src = edge_index[0].astype(jnp.int32)
    dst = edge_index[1].astype(jnp.int32)
    fill = N + (jnp.arange(EPAD - E, dtype=jnp.int32) % (NPAD - N))
    srcp = jnp.concatenate([src, fill])
    dstp = jnp.concatenate([dst, fill])
    src4 = srcp.reshape(NC, NS, NCH, CH)
    dst4 = dstp.reshape(NC, NS, NCH, CH)
    batch3 = jnp.concatenate(
        [batch.astype(jnp.int32), jnp.full((NPAD - N,), 1 << 20, jnp.int32)]
    ).reshape(NB, 1, BR)

    b1r = b1.reshape(1, D)
    b2r = b2.reshape(1, D)
    bcr = bc.reshape(1, FINAL)
    btr = bt.reshape(1, FINAL)
    bfr = bf.reshape(1, FINAL)
    Wfc = Wf[:CODE]
    Wft = Wf[CODE:]

    deg_kernel = _make_deg_kernel()
    agg_kernel = _make_agg_kernel()
    dst_deg = dstp.reshape(NC, NS, EPT)
    degp = deg_kernel(dst_deg).reshape(NC, NPAD, 1)
    g1, dinv16 = _tc1(x_pad, W1, degp)
    p1 = agg_kernel(g1, src4, dst4)
    g2 = _tc2(g1, p1, dinv16, b1r, W2)
    p2 = agg_kernel(g2, src4, dst4)
    code_prob, trans_prob, final_prob = _tc3(
        g2, p2, dinv16, b2r, batch3, code_x, Wc, bcr, Wt, btr, Wfc, Wft, bfr)
    return (code_prob, trans_prob, final_prob)


# speedup vs baseline: 4.2004x; 1.0490x over previous
import functools

import jax
import jax.numpy as jnp
from jax import lax
from jax.experimental import pallas as pl
from jax.experimental.pallas import tpu as pltpu
from jax.experimental.pallas import tpu_sc as plsc

N = 10000
E = 320000
B = 256
D = 128
CODE = 256
FINAL = 128

NPAD = 10240
BR = 512
NB = NPAD // BR

NC = 2
NS = 16
EPT = E // (NC * NS)
CH = 64
NCH = EPT // CH
TAIL = EPT - NCH * CH
NRING = 4
NOUT = NCH // NRING
RPT = NPAD // NS
DEGW = 16

_HIGH = jax.lax.Precision.HIGHEST


def _mesh():
    return plsc.VectorSubcoreMesh(core_axis_name="c", subcore_axis_name="s")


@functools.cache
def _make_deg_kernel():
    return functools.partial(
        pl.kernel,
        mesh=_mesh(),
        out_type=jax.ShapeDtypeStruct((NC, NPAD), jnp.float32),
        compiler_params=pltpu.CompilerParams(needs_layout_passes=False),
        scratch_types=[
            pltpu.VMEM((EPT,), jnp.int32),
            pltpu.VMEM((NPAD,), jnp.float32),
            pltpu.VMEM((NS, RPT), jnp.float32),
            pltpu.VMEM_SHARED((NS, NPAD), jnp.float32),
            pltpu.SemaphoreType.DMA,
        ],
    )(_deg_body)


def _deg_body(edge_hbm, deg_out, idx_v, hist_v, tmp_v, hists_sh, sem):
    cid = lax.axis_index("c")
    sid = lax.axis_index("s")
    wid = cid * NS + sid

    def zf(i, _):
        hist_v[pl.ds(i * 16, 16)] = jnp.zeros((16,), jnp.float32)
        return 0

    lax.fori_loop(0, NPAD // 16, zf, 0)
    pltpu.sync_copy(edge_hbm.at[pl.ds(E + wid * EPT, EPT)], idx_v)
    ones = jnp.full((16,), 1.0, jnp.float32)

    def body(k, _):
        for u in range(5):
            idx16 = idx_v[pl.ds((5 * k + u) * 16, 16)]
            plsc.addupdate_scatter(hist_v, [idx16], ones)
        return 0

    lax.fori_loop(0, EPT // 80, body, 0)
    pltpu.sync_copy(hist_v, hists_sh.at[sid])
    plsc.subcore_barrier()
    for t in range(NS):
        pltpu.async_copy(hists_sh.at[t, pl.ds(sid * RPT, RPT)],
                         tmp_v.at[t], sem)
    for t in range(NS):
        pltpu.make_async_copy(hists_sh.at[t, pl.ds(sid * RPT, RPT)],
                              tmp_v.at[t], sem).wait()

    def red(j, _):
        s = tmp_v[0, pl.ds(j * 16, 16)]
        for t in range(1, NS):
            s = s + tmp_v[t, pl.ds(j * 16, 16)]
        hist_v[pl.ds(j * 16, 16)] = s
        return 0

    lax.fori_loop(0, RPT // 16, red, 0)
    pltpu.sync_copy(hist_v.at[pl.ds(0, RPT)],
                    deg_out.at[cid, pl.ds(sid * RPT, RPT)])


@functools.cache
def _make_agg_kernel():
    return functools.partial(
        pl.kernel,
        mesh=_mesh(),
        out_type=jax.ShapeDtypeStruct((NC, NPAD, D), jnp.float32),
        scratch_types=[
            pltpu.VMEM((NRING, CH), jnp.int32),
            pltpu.VMEM((NRING, CH), jnp.int32),
            pltpu.VMEM((NRING, CH, D), jnp.float32),
            pltpu.VMEM((TAIL,), jnp.int32),
            pltpu.VMEM((TAIL,), jnp.int32),
            pltpu.VMEM((TAIL, D), jnp.float32),
            pltpu.VMEM_SHARED((NPAD, D), jnp.float32),
        ]
        + [pltpu.SemaphoreType.DMA] * (4 * NRING + 1),
    )(_agg_body)


def _agg_body(g_hbm, edge_hbm, out_hbm, isrc, idst, rows, tsrc, tdst, trows,
              acc_s, *sems):
    gsem = sems[0:NRING]
    ssem = sems[NRING:2 * NRING]
    isem = sems[2 * NRING:3 * NRING]
    dsem = sems[3 * NRING:4 * NRING]
    tsem = sems[4 * NRING]
    cid = lax.axis_index("c")
    sid = lax.axis_index("s")
    wid = cid * NS + sid
    base = wid * EPT

    def _src(c):
        return edge_hbm.at[pl.ds(base + c * CH, CH)]

    def _dst(c):
        return edge_hbm.at[pl.ds(E + base + c * CH, CH)]

    def zfill(i, _):
        for j in range(D // 16):
            rows[0, i, pl.ds(j * 16, 16)] = jnp.zeros((16,), jnp.float32)
        return 0

    lax.fori_loop(0, CH, zfill, 0)
    pltpu.async_copy(edge_hbm.at[pl.ds(base + NCH * CH, TAIL)], tsrc, tsem)
    pltpu.async_copy(edge_hbm.at[pl.ds(E + base + NCH * CH, TAIL)], tdst, tsem)
    for r in range(RPT // CH):
        pltpu.sync_copy(rows.at[0], acc_s.at[pl.ds(sid * RPT + r * CH, CH)])
    plsc.subcore_barrier()

    for j in range(NRING):
        pltpu.async_copy(_src(j), isrc.at[j], isem[j])
    for j in range(NRING - 1):
        pltpu.async_copy(_dst(j), idst.at[j], dsem[j])
        pltpu.make_async_copy(_src(j), isrc.at[j], isem[j]).wait()
        pltpu.async_copy(g_hbm.at[isrc.at[j]], rows.at[j], gsem[j])

    def body(o, _):
        for j in range(NRING):
            c = NRING * o + j
            b3 = (j + 3) % NRING
            pltpu.make_async_copy(g_hbm.at[isrc.at[j]], rows.at[j],
                                  gsem[j]).wait()
            pltpu.make_async_copy(_dst(c), idst.at[j], dsem[j]).wait()
            pltpu.async_copy(rows.at[j], acc_s.at[idst.at[j]], ssem[j],
                             add=True)

            @pl.when(o < NOUT - 1)
            def _pf():
                pltpu.async_copy(_src(c + NRING), isrc.at[j], isem[j])

            def _recycle():
                pltpu.make_async_copy(rows.at[b3], acc_s.at[idst.at[b3]],
                                      ssem[b3]).wait()
                pltpu.make_async_copy(_src(c), isrc.at[b3], isem[b3]).wait()
                pltpu.async_copy(g_hbm.at[isrc.at[b3]], rows.at[b3],
                                 gsem[b3])
                pltpu.async_copy(_dst(c + 3), idst.at[b3], dsem[b3])

            if j == 0:
                @pl.when(o > 0)
                def _w0():
                    pltpu.make_async_copy(rows.at[b3], acc_s.at[idst.at[b3]],
                                          ssem[b3]).wait()

                pltpu.make_async_copy(_src(c), isrc.at[b3], isem[b3]).wait()
                pltpu.async_copy(g_hbm.at[isrc.at[b3]], rows.at[b3],
                                 gsem[b3])
                pltpu.async_copy(_dst(c + 3), idst.at[b3], dsem[b3])
            else:
                @pl.when(o < NOUT - 1)
                def _wj():
                    _recycle()

        return 0

    lax.fori_loop(0, NOUT, body, 0)
    for j in range(NRING):
        pltpu.make_async_copy(rows.at[j], acc_s.at[idst.at[j]],
                              ssem[j]).wait()
    pltpu.make_async_copy(edge_hbm.at[pl.ds(base + NCH * CH, TAIL)],
                          tsrc, tsem).wait()
    pltpu.make_async_copy(edge_hbm.at[pl.ds(E + base + NCH * CH, TAIL)],
                          tdst, tsem).wait()
    pltpu.async_copy(g_hbm.at[tsrc], trows, tsem).wait()
    pltpu.sync_copy(trows, acc_s.at[tdst], add=True)
    plsc.subcore_barrier()
    pltpu.sync_copy(acc_s.at[pl.ds(sid * RPT, RPT)],
                    out_hbm.at[cid, pl.ds(sid * RPT, RPT)])


def _tc1_body(x_ref, w_ref, deg_ref, g_ref, dinv_ref):
    dtot = deg_ref[0] + deg_ref[1] + 1.0
    dinv = lax.rsqrt(jnp.maximum(dtot, 1.0))
    y = jnp.dot(x_ref[...], w_ref[...],
                preferred_element_type=jnp.float32, precision=_HIGH)
    g_ref[...] = y * dinv
    dinv_ref[...] = dinv


def _tc1(x_pad, W1, degp):
    return pl.pallas_call(
        _tc1_body,
        grid=(NB,),
        in_specs=[
            pl.BlockSpec((BR, D), lambda k: (k, 0)),
            pl.BlockSpec((D, D), lambda k: (0, 0)),
            pl.BlockSpec((NC, BR, 1), lambda k: (0, k, 0)),
        ],
        out_specs=[
            pl.BlockSpec((BR, D), lambda k: (k, 0)),
            pl.BlockSpec((BR, 1), lambda k: (k, 0)),
        ],
        out_shape=[
            jax.ShapeDtypeStruct((NPAD, D), jnp.float32),
            jax.ShapeDtypeStruct((NPAD, 1), jnp.float32),
        ],
    )(x_pad, W1, degp)


def _tc2_body(g_ref, p_ref, dinv_ref, b_ref, w_ref, o_ref):
    dinv = dinv_ref[...]
    s = p_ref[0] + p_ref[1] + g_ref[...]
    h = jnp.maximum(s * dinv + b_ref[...], 0.0)
    y = jnp.dot(h, w_ref[...],
                preferred_element_type=jnp.float32, precision=_HIGH)
    o_ref[...] = y * dinv


def _tc2(g1, p1, dinv16, b1, W2):
    return pl.pallas_call(
        _tc2_body,
        grid=(NB,),
        in_specs=[
            pl.BlockSpec((BR, D), lambda k: (k, 0)),
            pl.BlockSpec((NC, BR, D), lambda k: (0, k, 0)),
            pl.BlockSpec((BR, 1), lambda k: (k, 0)),
            pl.BlockSpec((1, D), lambda k: (0, 0)),
            pl.BlockSpec((D, D), lambda k: (0, 0)),
        ],
        out_specs=pl.BlockSpec((BR, D), lambda k: (k, 0)),
        out_shape=jax.ShapeDtypeStruct((NPAD, D), jnp.float32),
    )(g1, p1, dinv16, b1, W2)


def _log_softmax(z):
    m = jnp.max(z, axis=-1, keepdims=True)
    zs = z - m
    return zs - jnp.log(jnp.sum(jnp.exp(zs), axis=-1, keepdims=True))


def _tc3_body(g_ref, p_ref, dinv_ref, b2_ref, batch_ref, cx_ref,
              wc_ref, bc_ref, wt_ref, bt_ref, wfc_ref, wft_ref, bf_ref,
              o1_ref, o2_ref, o3_ref, sums, counts):
    k = pl.program_id(0)

    @pl.when(k == 0)
    def _init():
        sums[...] = jnp.zeros_like(sums)
        counts[...] = jnp.zeros_like(counts)

    dinv = dinv_ref[...]
    h2 = (p_ref[0] + p_ref[1] + g_ref[...]) * dinv + b2_ref[...]
    bvec = batch_ref[0]
    seg = lax.broadcasted_iota(jnp.int32, (B, BR), 0)
    onehot = (bvec == seg).astype(jnp.float32)
    sums[...] += jnp.dot(onehot, h2,
                         preferred_element_type=jnp.float32, precision=_HIGH)
    counts[...] += jnp.sum(onehot, axis=1, keepdims=True)

    @pl.when(k == NB - 1)
    def _final():
        cnt = jnp.maximum(counts[:, 0:1], 1.0)
        trans = sums[...] / cnt
        code = cx_ref[...]
        z1 = jnp.dot(code, wc_ref[...],
                     preferred_element_type=jnp.float32, precision=_HIGH) + bc_ref[...]
        o1_ref[...] = _log_softmax(z1)
        z2 = jnp.dot(trans, wt_ref[...],
                     preferred_element_type=jnp.float32, precision=_HIGH) + bt_ref[...]
        o2_ref[...] = _log_softmax(z2)
        z3 = (jnp.dot(code, wfc_ref[...],
                      preferred_element_type=jnp.float32, precision=_HIGH)
              + jnp.dot(trans, wft_ref[...],
                        preferred_element_type=jnp.float32, precision=_HIGH)
              + bf_ref[...])
        o3_ref[...] = _log_softmax(z3)


def _tc3(g2, p2, dinv16, b2, batch3, code_x, Wc, bc, Wt, bt, Wfc, Wft, bf):
    return pl.pallas_call(
        _tc3_body,
        grid=(NB,),
        in_specs=[
            pl.BlockSpec((BR, D), lambda k: (k, 0)),
            pl.BlockSpec((NC, BR, D), lambda k: (0, k, 0)),
            pl.BlockSpec((BR, 1), lambda k: (k, 0)),
            pl.BlockSpec((1, D), lambda k: (0, 0)),
            pl.BlockSpec((1, 1, BR), lambda k: (k, 0, 0)),
            pl.BlockSpec((B, CODE), lambda k: (0, 0)),
            pl.BlockSpec((CODE, FINAL), lambda k: (0, 0)),
            pl.BlockSpec((1, FINAL), lambda k: (0, 0)),
            pl.BlockSpec((D, FINAL), lambda k: (0, 0)),
            pl.BlockSpec((1, FINAL), lambda k: (0, 0)),
            pl.BlockSpec((CODE, FINAL), lambda k: (0, 0)),
            pl.BlockSpec((D, FINAL), lambda k: (0, 0)),
            pl.BlockSpec((1, FINAL), lambda k: (0, 0)),
        ],
        out_specs=[
            pl.BlockSpec((B, FINAL), lambda k: (0, 0)),
            pl.BlockSpec((B, FINAL), lambda k: (0, 0)),
            pl.BlockSpec((B, FINAL), lambda k: (0, 0)),
        ],
        out_shape=[
            jax.ShapeDtypeStruct((B, FINAL), jnp.float32),
            jax.ShapeDtypeStruct((B, FINAL), jnp.float32),
            jax.ShapeDtypeStruct((B, FINAL), jnp.float32),
        ],
        scratch_shapes=[
            pltpu.VMEM((B, FINAL), jnp.float32),
            pltpu.VMEM((B, FINAL), jnp.float32),
        ],
    )(g2, p2, dinv16, b2, batch3, code_x, Wc, bc, Wt, bt, Wfc, Wft, bf)


def kernel(x, edge_index, batch, code_x, W1, b1, W2, b2, Wc, bc, Wt, bt, Wf, bf):
    x_pad = jnp.pad(x, ((0, NPAD - N), (0, 0)))
    edges = edge_index.astype(jnp.int32).reshape(2 * E)
    batch3 = jnp.concatenate(
        [batch.astype(jnp.int32), jnp.full((NPAD - N,), 1 << 20, jnp.int32)]
    ).reshape(NB, 1, BR)

    b1r = b1.reshape(1, D)
    b2r = b2.reshape(1, D)
    bcr = bc.reshape(1, FINAL)
    btr = bt.reshape(1, FINAL)
    bfr = bf.reshape(1, FINAL)
    Wfc = Wf[:CODE]
    Wft = Wf[CODE:]

    deg_kernel = _make_deg_kernel()
    agg_kernel = _make_agg_kernel()
    degp = deg_kernel(edges).reshape(NC, NPAD, 1)
    g1, dinv16 = _tc1(x_pad, W1, degp)
    p1 = agg_kernel(g1, edges)
    g2 = _tc2(g1, p1, dinv16, b1r, W2)
    p2 = agg_kernel(g2, edges)
    code_prob, trans_prob, final_prob = _tc3(
        g2, p2, dinv16, b2r, batch3, code_x, Wc, bcr, Wt, btr, Wfc, Wft, bfr)
    return (code_prob, trans_prob, final_prob)
